# Initial kernel scaffold; baseline (speedup 1.0000x reference)
import functools

import jax
import jax.numpy as jnp
from jax import lax
from jax.experimental import pallas as pl
from jax.experimental.pallas import tpu as pltpu
from jax.experimental.pallas import tpu_sc as plsc

NORM_TERM = 13.0
KNOISE = 100
P = 112
T = 1024
D = 64
NW = 32
TW = T // NW


def _sc_body(kk_hbm, u_hbm, tgt_hbm, emb_hbm, noise_hbm, ap_hbm, aa_hbm,
             w_hbm, b_hbm,
             nsc_hbm, pn_hbm, tsc_hbm, ptn_hbm,
             kk_all, u_all, ap_v, aa_v, ns_all, pn_all, bias_all, scores,
             rows, emb_v, tv, trows, tb_v, ptn_v, tsc_v,
             sem0, sem1, sem2, semr0, semr1):
    c = lax.axis_index("c")
    s = lax.axis_index("s")
    wid = s * 2 + c
    base = wid * TW

    pltpu.sync_copy(kk_hbm.at[pl.ds(base, TW)], kk_all)
    pltpu.sync_copy(u_hbm.at[pl.ds(base, TW)], u_all)
    pltpu.sync_copy(emb_hbm.at[pl.ds(base, TW)], emb_v)
    pltpu.sync_copy(tgt_hbm.at[pl.ds(base, TW)], tv)

    def phase_a(lt, carry):
        cp0 = pltpu.make_async_copy(ap_hbm.at[kk_all.at[lt]], ap_v, sem0)
        cp1 = pltpu.make_async_copy(aa_hbm.at[kk_all.at[lt]], aa_v, sem1)
        cp0.start()
        cp1.start()
        cp0.wait()
        cp1.wait()
        for j in range(P // 16):
            sl = pl.ds(j * 16, 16)
            bsel = u_all[lt, sl] < ap_v[sl]
            ns_all[lt, sl] = jnp.where(bsel, kk_all[lt, sl], aa_v[sl])
        cp2 = pltpu.make_async_copy(noise_hbm.at[ns_all.at[lt]],
                                    pn_all.at[lt], sem0)
        cp3 = pltpu.make_async_copy(b_hbm.at[ns_all.at[lt]],
                                    bias_all.at[lt], sem1)
        cp2.start()
        cp3.start()
        cp2.wait()
        cp3.wait()
        return carry

    lax.fori_loop(0, TW, phase_a, 0)

    cpa = pltpu.make_async_copy(noise_hbm.at[tv], ptn_v, sem0)
    cpb = pltpu.make_async_copy(b_hbm.at[tv], tb_v, sem1)
    cpc = pltpu.make_async_copy(w_hbm.at[tv], trows, sem2)
    cpa.start()
    cpb.start()
    cpc.start()
    cpa.wait()
    cpb.wait()
    cpc.wait()
    for j in range(TW):
        m = emb_v[j, pl.ds(0, 16)] * trows[j, pl.ds(0, 16)]
        m = m + emb_v[j, pl.ds(16, 16)] * trows[j, pl.ds(16, 16)]
        m = m + emb_v[j, pl.ds(32, 16)] * trows[j, pl.ds(32, 16)]
        m = m + emb_v[j, pl.ds(48, 16)] * trows[j, pl.ds(48, 16)]
        tsc_v[j] = jnp.sum(m) + tb_v[j]

    def rows_copy(lt, b, sem):
        return pltpu.make_async_copy(w_hbm.at[ns_all.at[lt]], rows.at[b], sem)

    rsems = (semr0, semr1)
    rows_copy(0, 0, rsems[0]).start()

    def phase_b(i2, carry):
        for b in range(2):
            lt = i2 * 2 + b
            nxt = lt + 1

            @pl.when(nxt < TW)
            def _():
                rows_copy(nxt, 1 - b, rsems[1 - b]).start()

            rows_copy(lt, b, rsems[b]).wait()
            e0 = emb_v[lt, pl.ds(0, 16)]
            e1 = emb_v[lt, pl.ds(16, 16)]
            e2 = emb_v[lt, pl.ds(32, 16)]
            e3 = emb_v[lt, pl.ds(48, 16)]
            for i in range(P):
                m = rows[b, i, pl.ds(0, 16)] * e0
                m = m + rows[b, i, pl.ds(16, 16)] * e1
                m = m + rows[b, i, pl.ds(32, 16)] * e2
                m = m + rows[b, i, pl.ds(48, 16)] * e3
                scores[lt, i] = jnp.sum(m)
            for j in range(P // 16):
                sl = pl.ds(j * 16, 16)
                scores[lt, sl] = scores[lt, sl] + bias_all[lt, sl]
        return carry

    lax.fori_loop(0, TW // 2, phase_b, 0)

    pltpu.sync_copy(scores, nsc_hbm.at[pl.ds(base, TW)])
    pltpu.sync_copy(pn_all, pn_hbm.at[pl.ds(base, TW)])
    pltpu.sync_copy(tsc_v, tsc_hbm.at[pl.ds(base, TW)])
    pltpu.sync_copy(ptn_v, ptn_hbm.at[pl.ds(base, TW)])


_sc_call = functools.partial(
    pl.kernel,
    out_type=[
        jax.ShapeDtypeStruct((T, P), jnp.float32),
        jax.ShapeDtypeStruct((T, P), jnp.float32),
        jax.ShapeDtypeStruct((T,), jnp.float32),
        jax.ShapeDtypeStruct((T,), jnp.float32),
    ],
    mesh=plsc.VectorSubcoreMesh(core_axis_name="c", subcore_axis_name="s"),
    scratch_types=[
        pltpu.VMEM((TW, P), jnp.int32),
        pltpu.VMEM((TW, P), jnp.float32),
        pltpu.VMEM((P,), jnp.float32),
        pltpu.VMEM((P,), jnp.int32),
        pltpu.VMEM((TW, P), jnp.int32),
        pltpu.VMEM((TW, P), jnp.float32),
        pltpu.VMEM((TW, P), jnp.float32),
        pltpu.VMEM((TW, P), jnp.float32),
        pltpu.VMEM((2, P, D), jnp.float32),
        pltpu.VMEM((TW, D), jnp.float32),
        pltpu.VMEM((TW,), jnp.int32),
        pltpu.VMEM((TW, D), jnp.float32),
        pltpu.VMEM((TW,), jnp.float32),
        pltpu.VMEM((TW,), jnp.float32),
        pltpu.VMEM((TW,), jnp.float32),
        pltpu.SemaphoreType.DMA,
        pltpu.SemaphoreType.DMA,
        pltpu.SemaphoreType.DMA,
        pltpu.SemaphoreType.DMA,
        pltpu.SemaphoreType.DMA,
    ],
)(_sc_body)


def _tc_body(nsc_ref, pn_ref, tsc_ref, ptn_ref, out_ref):
    ns = nsc_ref[...]
    pn = pn_ref[...]
    pm = jnp.clip(jnp.exp(ns - NORM_TERM), 1e-9, 1.0)
    p = pm / (pm + 100.0 * pn)
    p = jnp.clip(p, 1e-12, 1.0 - 1e-12)
    lane = lax.broadcasted_iota(jnp.int32, ns.shape, 1)
    bce_n = jnp.where(lane < KNOISE, -jnp.log(1.0 - p), 0.0)
    ts = tsc_ref[...]
    ptn = ptn_ref[...]
    pmt = jnp.clip(jnp.exp(ts - NORM_TERM), 1e-9, 1.0)
    pt = pmt / (pmt + 100.0 * ptn)
    pt = jnp.clip(pt, 1e-12, 1.0 - 1e-12)
    bce_t = -jnp.log(pt)
    out_ref[0, 0] = (jnp.sum(bce_n) + jnp.sum(bce_t)) / float(T)


def _tc_call(nsc, pnv, tsc2, ptn2):
    return pl.pallas_call(
        _tc_body,
        out_shape=jax.ShapeDtypeStruct((1, 1), jnp.float32),
        out_specs=pl.BlockSpec(memory_space=pltpu.SMEM),
    )(nsc, pnv, tsc2, ptn2)


def kernel(target, emb, noise, alias_prob, alias_alias, weight, bias):
    B, N = target.shape
    V = noise.shape[0]
    kk = jax.random.randint(jax.random.key(42), (B, N, KNOISE), 0, V)
    u = jax.random.uniform(jax.random.key(43), (B, N, KNOISE), jnp.float32)
    kk2 = jnp.pad(kk.reshape(T, KNOISE).astype(jnp.int32),
                  ((0, 0), (0, P - KNOISE)))
    u2 = jnp.pad(u.reshape(T, KNOISE), ((0, 0), (0, P - KNOISE)),
                 constant_values=2.0)
    tgt = target.reshape(T).astype(jnp.int32)
    embf = emb.reshape(T, D)
    aa = alias_alias.astype(jnp.int32)
    nsc, pnv, tsc, ptn = _sc_call(kk2, u2, tgt, embf, noise, alias_prob, aa,
                                  weight, bias)
    loss = _tc_call(nsc, pnv, tsc.reshape(8, T // 8), ptn.reshape(8, T // 8))
    return loss[0, 0]


# speedup vs baseline: 1.3856x; 1.3856x over previous
import functools

import jax
import jax.numpy as jnp
from jax import lax
from jax.experimental import pallas as pl
from jax.experimental.pallas import tpu as pltpu
from jax.experimental.pallas import tpu_sc as plsc

NORM_TERM = 13.0
KNOISE = 100
P = 112
T = 1024
D = 64
NW = 32
TW = T // NW


def _sc_body(kk_hbm, u_hbm, tgt_hbm, emb_hbm, noise_hbm, ap_hbm, aa_hbm,
             w_hbm, b_hbm,
             nsc_hbm, pn_hbm, tsc_hbm, ptn_hbm,
             kk_all, u_all, ap_v, aa_v, ns_all, pn_all, bias_all, scores,
             rows, emb_v, tv, trows, tb_v, ptn_v, tsc_v,
             sem0, sem1, sem2, semr0, semr1):
    c = lax.axis_index("c")
    s = lax.axis_index("s")
    wid = s * 2 + c
    base = wid * TW

    pltpu.sync_copy(kk_hbm.at[pl.ds(base, TW)], kk_all)
    pltpu.sync_copy(u_hbm.at[pl.ds(base, TW)], u_all)
    pltpu.sync_copy(emb_hbm.at[pl.ds(base, TW)], emb_v)
    pltpu.sync_copy(tgt_hbm.at[pl.ds(base, TW)], tv)

    def phase_a(lt, carry):
        cp0 = pltpu.make_async_copy(ap_hbm.at[kk_all.at[lt]], ap_v, sem0)
        cp1 = pltpu.make_async_copy(aa_hbm.at[kk_all.at[lt]], aa_v, sem1)
        cp0.start()
        cp1.start()
        cp0.wait()
        cp1.wait()
        for j in range(P // 16):
            sl = pl.ds(j * 16, 16)
            bsel = u_all[lt, sl] < ap_v[sl]
            ns_all[lt, sl] = jnp.where(bsel, kk_all[lt, sl], aa_v[sl])
        cp2 = pltpu.make_async_copy(noise_hbm.at[ns_all.at[lt]],
                                    pn_all.at[lt], sem0)
        cp3 = pltpu.make_async_copy(b_hbm.at[ns_all.at[lt]],
                                    bias_all.at[lt], sem1)
        cp2.start()
        cp3.start()
        cp2.wait()
        cp3.wait()
        return carry

    lax.fori_loop(0, TW, phase_a, 0)

    cpa = pltpu.make_async_copy(noise_hbm.at[tv], ptn_v, sem0)
    cpb = pltpu.make_async_copy(b_hbm.at[tv], tb_v, sem1)
    cpc = pltpu.make_async_copy(w_hbm.at[tv], trows, sem2)
    cpa.start()
    cpb.start()
    cpc.start()
    cpa.wait()
    cpb.wait()
    cpc.wait()
    iota16 = lax.iota(jnp.int32, 16)
    for tg in range(TW // 16):
        tok16 = iota16 + (tg * 16)
        acc = jnp.zeros((16,), jnp.float32)
        for d in range(D):
            dfull = jnp.full((16,), d, jnp.int32)
            wv = plsc.load_gather(trows, [tok16, dfull])
            ev = plsc.load_gather(emb_v, [tok16, dfull])
            acc = acc + wv * ev
        tsc_v[pl.ds(tg * 16, 16)] = acc + tb_v[pl.ds(tg * 16, 16)]

    def rows_copy(lt, b, sem):
        return pltpu.make_async_copy(w_hbm.at[ns_all.at[lt]], rows.at[b], sem)

    rsems = (semr0, semr1)
    rows_copy(0, 0, rsems[0]).start()
    sidx = [iota16 + (g * 16) for g in range(P // 16)]
    dfulls = [jnp.full((16,), d, jnp.int32) for d in range(D)]

    def phase_b(i2, carry):
        for b in range(2):
            lt = i2 * 2 + b
            nxt = lt + 1

            @pl.when(nxt < TW)
            def _():
                rows_copy(nxt, 1 - b, rsems[1 - b]).start()

            rows_copy(lt, b, rsems[b]).wait()
            evs = [emb_v[lt, pl.ds(16 * j, 16)] for j in range(D // 16)]
            rowsb = rows.at[b]
            accs = [jnp.zeros((16,), jnp.float32) for _ in range(P // 16)]
            for d in range(D):
                e_d = evs[d // 16][d % 16]
                for g in range(P // 16):
                    col = plsc.load_gather(rowsb, [sidx[g], dfulls[d]])
                    accs[g] = accs[g] + col * e_d
            for g in range(P // 16):
                sl = pl.ds(g * 16, 16)
                scores[lt, sl] = accs[g] + bias_all[lt, sl]
        return carry

    lax.fori_loop(0, TW // 2, phase_b, 0)

    pltpu.sync_copy(scores, nsc_hbm.at[pl.ds(base, TW)])
    pltpu.sync_copy(pn_all, pn_hbm.at[pl.ds(base, TW)])
    pltpu.sync_copy(tsc_v, tsc_hbm.at[pl.ds(base, TW)])
    pltpu.sync_copy(ptn_v, ptn_hbm.at[pl.ds(base, TW)])


_sc_call = functools.partial(
    pl.kernel,
    out_type=[
        jax.ShapeDtypeStruct((T, P), jnp.float32),
        jax.ShapeDtypeStruct((T, P), jnp.float32),
        jax.ShapeDtypeStruct((T,), jnp.float32),
        jax.ShapeDtypeStruct((T,), jnp.float32),
    ],
    mesh=plsc.VectorSubcoreMesh(core_axis_name="c", subcore_axis_name="s"),
    compiler_params=pltpu.CompilerParams(use_tc_tiling_on_sc=False,
                                         needs_layout_passes=False),
    scratch_types=[
        pltpu.VMEM((TW, P), jnp.int32),
        pltpu.VMEM((TW, P), jnp.float32),
        pltpu.VMEM((P,), jnp.float32),
        pltpu.VMEM((P,), jnp.int32),
        pltpu.VMEM((TW, P), jnp.int32),
        pltpu.VMEM((TW, P), jnp.float32),
        pltpu.VMEM((TW, P), jnp.float32),
        pltpu.VMEM((TW, P), jnp.float32),
        pltpu.VMEM((2, P, D), jnp.float32),
        pltpu.VMEM((TW, D), jnp.float32),
        pltpu.VMEM((TW,), jnp.int32),
        pltpu.VMEM((TW, D), jnp.float32),
        pltpu.VMEM((TW,), jnp.float32),
        pltpu.VMEM((TW,), jnp.float32),
        pltpu.VMEM((TW,), jnp.float32),
        pltpu.SemaphoreType.DMA,
        pltpu.SemaphoreType.DMA,
        pltpu.SemaphoreType.DMA,
        pltpu.SemaphoreType.DMA,
        pltpu.SemaphoreType.DMA,
    ],
)(_sc_body)


def _tc_body(nsc_ref, pn_ref, tsc_ref, ptn_ref, out_ref):
    ns = nsc_ref[...]
    pn = pn_ref[...]
    pm = jnp.clip(jnp.exp(ns - NORM_TERM), 1e-9, 1.0)
    p = pm / (pm + 100.0 * pn)
    p = jnp.clip(p, 1e-12, 1.0 - 1e-12)
    lane = lax.broadcasted_iota(jnp.int32, ns.shape, 1)
    bce_n = jnp.where(lane < KNOISE, -jnp.log(1.0 - p), 0.0)
    ts = tsc_ref[...]
    ptn = ptn_ref[...]
    pmt = jnp.clip(jnp.exp(ts - NORM_TERM), 1e-9, 1.0)
    pt = pmt / (pmt + 100.0 * ptn)
    pt = jnp.clip(pt, 1e-12, 1.0 - 1e-12)
    bce_t = -jnp.log(pt)
    out_ref[0, 0] = (jnp.sum(bce_n) + jnp.sum(bce_t)) / float(T)


def _tc_call(nsc, pnv, tsc2, ptn2):
    return pl.pallas_call(
        _tc_body,
        out_shape=jax.ShapeDtypeStruct((1, 1), jnp.float32),
        out_specs=pl.BlockSpec(memory_space=pltpu.SMEM),
    )(nsc, pnv, tsc2, ptn2)


def kernel(target, emb, noise, alias_prob, alias_alias, weight, bias):
    B, N = target.shape
    V = noise.shape[0]
    kk = jax.random.randint(jax.random.key(42), (B, N, KNOISE), 0, V)
    u = jax.random.uniform(jax.random.key(43), (B, N, KNOISE), jnp.float32)
    kk2 = jnp.pad(kk.reshape(T, KNOISE).astype(jnp.int32),
                  ((0, 0), (0, P - KNOISE)))
    u2 = jnp.pad(u.reshape(T, KNOISE), ((0, 0), (0, P - KNOISE)),
                 constant_values=2.0)
    tgt = target.reshape(T).astype(jnp.int32)
    embf = emb.reshape(T, D)
    aa = alias_alias.astype(jnp.int32)
    nsc, pnv, tsc, ptn = _sc_call(kk2, u2, tgt, embf, noise, alias_prob, aa,
                                  weight, bias)
    loss = _tc_call(nsc, pnv, tsc.reshape(8, T // 8), ptn.reshape(8, T // 8))
    return loss[0, 0]
